# baseline (speedup 1.0000x reference)
import functools

import jax
import jax.numpy as jnp
import numpy as np
from jax.experimental import pallas as pl
from jax.experimental.pallas import tpu as pltpu

NC = 5


def _layernorm(h, g, b, eps=1e-5):
    mu = jnp.mean(h, axis=-1, keepdims=True)
    var = jnp.mean((h - mu) ** 2, axis=-1, keepdims=True)
    return g * (h - mu) * jax.lax.rsqrt(var + eps) + b


def _prologue_kernel(x_ref, Win_ref, Wskip_ref, bskip_ref, u1_ref, skip_ref):
    xb = x_ref[...]
    u1 = jnp.dot(xb, Win_ref[...], preferred_element_type=jnp.float32)
    u1_ref[...] = u1.astype(jnp.bfloat16)
    sk = jnp.dot(xb, Wskip_ref[...], preferred_element_type=jnp.float32)
    skip_ref[...] = 0.1 * (sk + bskip_ref[...])


def _phase_a_kernel(*refs, br, cw):
    (a_ref, u1_ref, bin_ref, gin_ref, bein_ref, Wh_ref,
     adjq_ref, h1_ref, u2_ref, acc2_ref, u2_scr, u2_pend) = refs
    j = pl.program_id(0)
    g = cw // br

    @pl.when(j == 0)
    def _init():
        u2_scr[...] = jnp.zeros_like(u2_scr)

    @pl.when((j > 0) & (j % g == 0))
    def _fill():
        u2_scr[pl.ds((j // g - 1) * cw, cw), :] = u2_pend[...]

    q = a_ref[...].astype(jnp.bfloat16)
    for c in range(NC):
        adjq_ref[c] = q[:, c * cw:(c + 1) * cw]
    acc1 = jnp.dot(q, u1_ref[...], preferred_element_type=jnp.float32)
    acc2 = jnp.dot(q, u2_scr[...], preferred_element_type=jnp.float32)
    h1 = _layernorm(acc1 + bin_ref[...], gin_ref[...], bein_ref[...])
    h1 = jnp.maximum(h1, 0.0)
    h1_ref[...] = h1
    acc2_ref[...] = acc2
    u2j = jnp.dot(h1, Wh_ref[...],
                  preferred_element_type=jnp.float32).astype(jnp.bfloat16)
    u2_ref[...] = u2j
    u2_pend[pl.ds((j % g) * br, br), :] = u2j


def _phase_bc_kernel(sched_ref, adjq_ref, u2_ref, bh_ref, gh_ref, beh_ref,
                     h1_ref, acc2in_ref, Wout_ref, bout_ref, gout_ref,
                     beout_ref, skip_ref, out_ref,
                     acc2_scr, u3_scr, out_scr, *, cw, hh, cc):
    t = pl.program_id(0)
    jr = sched_ref[0, t]
    b = sched_ref[1, t]
    ty = sched_ref[2, t]
    firstb = sched_ref[3, t] == 1
    orow = sched_ref[5, t]
    q = adjq_ref[0]

    @pl.when(t == 0)
    def _zero():
        out_scr[...] = jnp.zeros_like(out_scr)

    @pl.when(firstb)
    def _init():
        acc2_scr[...] = acc2in_ref[...]

    @pl.when(ty == 0)
    def _b_offdiag():
        acc2_scr[...] += jnp.dot(q, u2_ref[pl.ds(b * cw, cw), :],
                                 preferred_element_type=jnp.float32)
        out_scr[pl.ds(jr * cw, cw), :] += jnp.dot(
            q, u3_scr[pl.ds(b * cw, cw), :],
            preferred_element_type=jnp.float32)

    @pl.when(ty == 1)
    def _b_diag():
        acc2_scr[...] += jnp.dot(q, u2_ref[pl.ds(b * cw, cw), :],
                                 preferred_element_type=jnp.float32)
        h2 = _layernorm(acc2_scr[...] + bh_ref[...], gh_ref[...],
                        beh_ref[...])
        h = jnp.maximum(h2, 0.0) + h1_ref[...]
        u3j = jnp.dot(h, Wout_ref[...],
                      preferred_element_type=jnp.float32).astype(jnp.bfloat16)
        u3_scr[pl.ds(jr * cw, cw), :] = u3j
        out_scr[pl.ds(jr * cw, cw), :] += jnp.dot(
            q, u3j, preferred_element_type=jnp.float32)

    @pl.when((ty == 2) | (ty == 3))
    def _c_acc():
        out_scr[pl.ds(jr * cw, cw), :] += jnp.dot(
            q, u3_scr[pl.ds(b * cw, cw), :],
            preferred_element_type=jnp.float32)

    @pl.when((ty == 3) | (ty == 4))
    def _finalize():
        o = _layernorm(out_scr[pl.ds(orow * cw, cw), :] + bout_ref[...],
                       gout_ref[...], beout_ref[...])
        out_ref[...] = o + skip_ref[...]


def kernel(x, adj, W_in, b_in, g_in, be_in, W_h, b_h, g_h, be_h,
           W_out, b_out, g_out, be_out, W_skip, b_skip):
    N, F = x.shape
    H = W_in.shape[1]
    C = W_out.shape[1]
    CW = N // NC
    BRA = CW // 10 if (CW % 10 == 0 and (CW // 10) % 8 == 0) else CW
    MB = NC
    _cp = pltpu.CompilerParams(vmem_limit_bytes=128 * 1024 * 1024)

    b_in2 = b_in.reshape(1, H)
    g_in2 = g_in.reshape(1, H)
    be_in2 = be_in.reshape(1, H)
    b_h2 = b_h.reshape(1, H)
    g_h2 = g_h.reshape(1, H)
    be_h2 = be_h.reshape(1, H)
    b_out2 = b_out.reshape(1, C)
    g_out2 = g_out.reshape(1, C)
    be_out2 = be_out.reshape(1, C)
    b_skip2 = b_skip.reshape(1, C)

    full = lambda shape: pl.BlockSpec(shape, lambda *a: (0,) * len(shape))

    u1, skip = pl.pallas_call(
        _prologue_kernel,
        grid=(1,),
        in_specs=[full((N, F)), full((F, H)), full((F, C)), full((1, C))],
        out_specs=[full((N, H)), full((N, C))],
        out_shape=[
            jax.ShapeDtypeStruct((N, H), jnp.bfloat16),
            jax.ShapeDtypeStruct((N, C), jnp.float32),
        ],
    )(x, W_in, W_skip, b_skip2)

    rowa = lambda w: pl.BlockSpec((BRA, w), lambda i: (i, 0))
    adjq, h1, u2, acc2p = pl.pallas_call(
        functools.partial(_phase_a_kernel, br=BRA, cw=CW),
        grid=(N // BRA,),
        in_specs=[rowa(N), full((N, H)), full((1, H)), full((1, H)),
                  full((1, H)), full((H, H))],
        out_specs=[pl.BlockSpec((NC, BRA, CW), lambda i: (0, i, 0)),
                   rowa(H), rowa(H), rowa(H)],
        out_shape=[
            jax.ShapeDtypeStruct((NC, N, CW), jnp.bfloat16),
            jax.ShapeDtypeStruct((N, H), jnp.float32),
            jax.ShapeDtypeStruct((N, H), jnp.bfloat16),
            jax.ShapeDtypeStruct((N, H), jnp.float32),
        ],
        scratch_shapes=[pltpu.VMEM((N, H), jnp.bfloat16),
                        pltpu.VMEM((CW, H), jnp.bfloat16)],
        compiler_params=_cp,
    )(adj, u1, b_in2, g_in2, be_in2, W_h)

    seq = []
    for j in range(MB - 1, -1, -1):
        bs = list(range(j + 1, MB)) + [j]
        for k, b in enumerate(bs):
            seq.append([j, b, 1 if b == j else 0, 1 if k == 0 else 0, j, -1])
        for jc in range(j + 1, MB):
            seq.append([jc, j, 3 if j == 0 else 2, 0, j, jc if j == 0 else -1])
    seq.append([seq[-1][0], seq[-1][1], 4, 0, 0, 0])
    first_fin = next(r[5] for r in seq if r[5] >= 0)
    cur = first_fin
    for r in seq:
        if r[5] >= 0:
            cur = r[5]
        else:
            r[5] = cur
    sched = jnp.asarray(np.array(seq, dtype=np.int32).T)
    TT = len(seq)

    tile = pl.BlockSpec((1, CW, CW), lambda t, s: (s[1, t], s[0, t], 0))
    rowb = lambda w: pl.BlockSpec((CW, w), lambda t, s: (s[4, t], 0))
    rowo = lambda w: pl.BlockSpec((CW, w), lambda t, s: (s[5, t], 0))
    fullp = lambda shape: pl.BlockSpec(shape, lambda t, s: (0,) * len(shape))

    out = pl.pallas_call(
        functools.partial(_phase_bc_kernel, cw=CW, hh=H, cc=C),
        grid_spec=pltpu.PrefetchScalarGridSpec(
            num_scalar_prefetch=1,
            grid=(TT,),
            in_specs=[tile, fullp((N, H)), fullp((1, H)), fullp((1, H)),
                      fullp((1, H)), rowb(H), rowb(H), fullp((H, C)),
                      fullp((1, C)), fullp((1, C)), fullp((1, C)), rowo(C)],
            out_specs=rowo(C),
            scratch_shapes=[pltpu.VMEM((CW, H), jnp.float32),
                            pltpu.VMEM((N, C), jnp.bfloat16),
                            pltpu.VMEM((N, C), jnp.float32)],
        ),
        out_shape=jax.ShapeDtypeStruct((N, C), jnp.float32),
        compiler_params=_cp,
    )(sched, adjq, u2, b_h2, g_h2, be_h2, h1, acc2p, W_out,
      b_out2, g_out2, be_out2, skip)

    return out


# speedup vs baseline: 1.0597x; 1.0597x over previous
import functools

import jax
import jax.numpy as jnp
import numpy as np
from jax.experimental import pallas as pl
from jax.experimental.pallas import tpu as pltpu

NC = 5


def _layernorm(h, g, b, eps=1e-5):
    mu = jnp.mean(h, axis=-1, keepdims=True)
    var = jnp.mean((h - mu) ** 2, axis=-1, keepdims=True)
    return g * (h - mu) * jax.lax.rsqrt(var + eps) + b


def _prologue_kernel(x_ref, Win_ref, Wskip_ref, bskip_ref, u1_ref, skip_ref):
    xb = x_ref[...]
    u1 = jnp.dot(xb, Win_ref[...], preferred_element_type=jnp.float32)
    u1_ref[...] = u1.astype(jnp.bfloat16)
    sk = jnp.dot(xb, Wskip_ref[...], preferred_element_type=jnp.float32)
    skip_ref[...] = 0.1 * (sk + bskip_ref[...])


def _phase_a_kernel(*refs, br, cw):
    (a_ref, u1_ref, bin_ref, gin_ref, bein_ref, Wh_ref,
     adjq_ref, h1_ref, u2_ref, acc2_ref, u2_scr, u2_pend) = refs
    j = pl.program_id(0)
    g = cw // br

    @pl.when(j == 0)
    def _init():
        u2_scr[...] = jnp.zeros_like(u2_scr)

    @pl.when((j > 0) & (j % g == 0))
    def _fill():
        u2_scr[pl.ds((j // g - 1) * cw, cw), :] = u2_pend[...]

    q = a_ref[...].astype(jnp.bfloat16)
    for c in range(NC):
        adjq_ref[c] = q[:, c * cw:(c + 1) * cw]
    acc1 = jnp.dot(q, u1_ref[...], preferred_element_type=jnp.float32)
    acc2 = jnp.dot(q, u2_scr[...], preferred_element_type=jnp.float32)
    h1 = _layernorm(acc1 + bin_ref[...], gin_ref[...], bein_ref[...])
    h1 = jnp.maximum(h1, 0.0)
    h1_ref[...] = h1
    acc2_ref[...] = acc2
    u2j = jnp.dot(h1, Wh_ref[...],
                  preferred_element_type=jnp.float32).astype(jnp.bfloat16)
    u2_ref[...] = u2j
    u2_pend[pl.ds((j % g) * br, br), :] = u2j


def _phase_bc_kernel(sched_ref, adjq_ref, u2_ref, bh_ref, gh_ref, beh_ref,
                     h1_ref, acc2in_ref, Wout_ref, bout_ref, gout_ref,
                     beout_ref, skip_ref, out_ref,
                     acc2_scr, u3_scr, out_scr, *, cw, hh, cc):
    t = pl.program_id(0)
    jr = sched_ref[0, t]
    b = sched_ref[1, t]
    ty = sched_ref[2, t]
    firstb = sched_ref[3, t] == 1
    orow = sched_ref[5, t]
    q = adjq_ref[0]

    @pl.when(t == 0)
    def _zero():
        out_scr[...] = jnp.zeros_like(out_scr)

    @pl.when(firstb)
    def _init():
        acc2_scr[...] = acc2in_ref[...]

    @pl.when(ty == 0)
    def _b_offdiag():
        rhs = jnp.concatenate(
            [u2_ref[pl.ds(b * cw, cw), :], u3_scr[pl.ds(b * cw, cw), :]],
            axis=1)
        r = jnp.dot(q, rhs, preferred_element_type=jnp.float32)
        acc2_scr[...] += r[:, :hh]
        out_scr[pl.ds(jr * cw, cw), :] += r[:, hh:hh + cc]

    @pl.when(ty == 1)
    def _b_diag():
        acc2_scr[...] += jnp.dot(q, u2_ref[pl.ds(b * cw, cw), :],
                                 preferred_element_type=jnp.float32)
        h2 = _layernorm(acc2_scr[...] + bh_ref[...], gh_ref[...],
                        beh_ref[...])
        h = jnp.maximum(h2, 0.0) + h1_ref[...]
        u3j = jnp.dot(h, Wout_ref[...],
                      preferred_element_type=jnp.float32).astype(jnp.bfloat16)
        u3_scr[pl.ds(jr * cw, cw), :] = u3j
        out_scr[pl.ds(jr * cw, cw), :] += jnp.dot(
            q, u3j, preferred_element_type=jnp.float32)

    @pl.when((ty == 2) | (ty == 3))
    def _c_acc():
        out_scr[pl.ds(jr * cw, cw), :] += jnp.dot(
            q, u3_scr[pl.ds(b * cw, cw), :],
            preferred_element_type=jnp.float32)

    @pl.when((ty == 3) | (ty == 4))
    def _finalize():
        o = _layernorm(out_scr[pl.ds(orow * cw, cw), :] + bout_ref[...],
                       gout_ref[...], beout_ref[...])
        out_ref[...] = o + skip_ref[...]


def kernel(x, adj, W_in, b_in, g_in, be_in, W_h, b_h, g_h, be_h,
           W_out, b_out, g_out, be_out, W_skip, b_skip):
    N, F = x.shape
    H = W_in.shape[1]
    C = W_out.shape[1]
    CW = N // NC
    BRA = CW // 10 if (CW % 10 == 0 and (CW // 10) % 8 == 0) else CW
    MB = NC
    _cp = pltpu.CompilerParams(vmem_limit_bytes=128 * 1024 * 1024)

    b_in2 = b_in.reshape(1, H)
    g_in2 = g_in.reshape(1, H)
    be_in2 = be_in.reshape(1, H)
    b_h2 = b_h.reshape(1, H)
    g_h2 = g_h.reshape(1, H)
    be_h2 = be_h.reshape(1, H)
    b_out2 = b_out.reshape(1, C)
    g_out2 = g_out.reshape(1, C)
    be_out2 = be_out.reshape(1, C)
    b_skip2 = b_skip.reshape(1, C)

    full = lambda shape: pl.BlockSpec(shape, lambda *a: (0,) * len(shape))

    u1, skip = pl.pallas_call(
        _prologue_kernel,
        grid=(1,),
        in_specs=[full((N, F)), full((F, H)), full((F, C)), full((1, C))],
        out_specs=[full((N, H)), full((N, C))],
        out_shape=[
            jax.ShapeDtypeStruct((N, H), jnp.bfloat16),
            jax.ShapeDtypeStruct((N, C), jnp.float32),
        ],
    )(x, W_in, W_skip, b_skip2)

    rowa = lambda w: pl.BlockSpec((BRA, w), lambda i: (i, 0))
    adjq, h1, u2, acc2p = pl.pallas_call(
        functools.partial(_phase_a_kernel, br=BRA, cw=CW),
        grid=(N // BRA,),
        in_specs=[rowa(N), full((N, H)), full((1, H)), full((1, H)),
                  full((1, H)), full((H, H))],
        out_specs=[pl.BlockSpec((NC, BRA, CW), lambda i: (0, i, 0)),
                   rowa(H), rowa(H), rowa(H)],
        out_shape=[
            jax.ShapeDtypeStruct((NC, N, CW), jnp.bfloat16),
            jax.ShapeDtypeStruct((N, H), jnp.float32),
            jax.ShapeDtypeStruct((N, H), jnp.bfloat16),
            jax.ShapeDtypeStruct((N, H), jnp.float32),
        ],
        scratch_shapes=[pltpu.VMEM((N, H), jnp.bfloat16),
                        pltpu.VMEM((CW, H), jnp.bfloat16)],
        compiler_params=_cp,
    )(adj, u1, b_in2, g_in2, be_in2, W_h)

    seq = []
    for j in range(MB - 1, -1, -1):
        bs = list(range(j + 1, MB)) + [j]
        for k, b in enumerate(bs):
            seq.append([j, b, 1 if b == j else 0, 1 if k == 0 else 0, j, -1])
        for jc in range(j + 1, MB):
            seq.append([jc, j, 3 if j == 0 else 2, 0, j, jc if j == 0 else -1])
    seq.append([seq[-1][0], seq[-1][1], 4, 0, 0, 0])
    first_fin = next(r[5] for r in seq if r[5] >= 0)
    cur = first_fin
    for r in seq:
        if r[5] >= 0:
            cur = r[5]
        else:
            r[5] = cur
    sched = jnp.asarray(np.array(seq, dtype=np.int32).T)
    TT = len(seq)

    tile = pl.BlockSpec((1, CW, CW), lambda t, s: (s[1, t], s[0, t], 0))
    rowb = lambda w: pl.BlockSpec((CW, w), lambda t, s: (s[4, t], 0))
    rowo = lambda w: pl.BlockSpec((CW, w), lambda t, s: (s[5, t], 0))
    fullp = lambda shape: pl.BlockSpec(shape, lambda t, s: (0,) * len(shape))

    out = pl.pallas_call(
        functools.partial(_phase_bc_kernel, cw=CW, hh=H, cc=C),
        grid_spec=pltpu.PrefetchScalarGridSpec(
            num_scalar_prefetch=1,
            grid=(TT,),
            in_specs=[tile, fullp((N, H)), fullp((1, H)), fullp((1, H)),
                      fullp((1, H)), rowb(H), rowb(H), fullp((H, C)),
                      fullp((1, C)), fullp((1, C)), fullp((1, C)), rowo(C)],
            out_specs=rowo(C),
            scratch_shapes=[pltpu.VMEM((CW, H), jnp.float32),
                            pltpu.VMEM((N, C), jnp.bfloat16),
                            pltpu.VMEM((N, C), jnp.float32)],
        ),
        out_shape=jax.ShapeDtypeStruct((N, C), jnp.float32),
        compiler_params=_cp,
    )(sched, adjq, u2, b_h2, g_h2, be_h2, h1, acc2p, W_out,
      b_out2, g_out2, be_out2, skip)

    return out


# speedup vs baseline: 1.0669x; 1.0068x over previous
import functools

import jax
import jax.numpy as jnp
import numpy as np
from jax.experimental import pallas as pl
from jax.experimental.pallas import tpu as pltpu

NC = 5


def _layernorm(h, g, b, eps=1e-5):
    mu = jnp.mean(h, axis=-1, keepdims=True)
    var = jnp.mean((h - mu) ** 2, axis=-1, keepdims=True)
    return g * (h - mu) * jax.lax.rsqrt(var + eps) + b


def _prologue_kernel(x_ref, Win_ref, Wskip_ref, bskip_ref, u1_ref, skip_ref):
    xb = x_ref[...]
    u1 = jnp.dot(xb, Win_ref[...], preferred_element_type=jnp.float32)
    u1_ref[...] = u1.astype(jnp.bfloat16)
    sk = jnp.dot(xb, Wskip_ref[...], preferred_element_type=jnp.float32)
    skip_ref[...] = 0.1 * (sk + bskip_ref[...])


def _phase_a_kernel(*refs, br, cw):
    (a_ref, u1_ref, bin_ref, gin_ref, bein_ref, Wh_ref,
     adjq_ref, h1_ref, u2_ref, acc2_ref, u2_scr, u2_pend) = refs
    j = pl.program_id(0)
    g = cw // br

    @pl.when(j == 0)
    def _init():
        u2_scr[...] = jnp.zeros_like(u2_scr)

    @pl.when((j > 0) & (j % g == 0))
    def _fill():
        u2_scr[pl.ds((j // g - 1) * cw, cw), :] = u2_pend[...]

    q = a_ref[...].astype(jnp.bfloat16)
    for c in range(NC):
        adjq_ref[c] = q[:, c * cw:(c + 1) * cw]
    acc1 = jnp.dot(q, u1_ref[...], preferred_element_type=jnp.float32)
    acc2 = jnp.dot(q, u2_scr[...], preferred_element_type=jnp.float32)
    h1 = _layernorm(acc1 + bin_ref[...], gin_ref[...], bein_ref[...])
    h1 = jnp.maximum(h1, 0.0)
    h1_ref[...] = h1
    acc2_ref[...] = acc2
    u2j = jnp.dot(h1, Wh_ref[...],
                  preferred_element_type=jnp.float32).astype(jnp.bfloat16)
    u2_ref[...] = u2j
    u2_pend[pl.ds((j % g) * br, br), :] = u2j


def _phase_bc_kernel(sched_ref, adjq_ref, u2_ref, bh_ref, gh_ref, beh_ref,
                     h1_ref, acc2in_ref, Wout_ref, bout_ref, gout_ref,
                     beout_ref, skip_ref, out_ref,
                     acc2_scr, u3_scr, out_scr, *, cw, hh, cc):
    t = pl.program_id(0)
    jr = sched_ref[0, t]
    b = sched_ref[1, t]
    ty = sched_ref[2, t]
    firstb = sched_ref[3, t] == 1
    orow = sched_ref[5, t]
    q = adjq_ref[0]

    @pl.when(t == 0)
    def _zero():
        out_scr[...] = jnp.zeros_like(out_scr)

    @pl.when(firstb)
    def _init():
        acc2_scr[...] = acc2in_ref[...]

    @pl.when(ty == 0)
    def _b_offdiag():
        rhs = jnp.concatenate(
            [u2_ref[pl.ds(b * cw, cw), :], u3_scr[pl.ds(b * cw, cw), :]],
            axis=1)
        r = jnp.dot(q, rhs, preferred_element_type=jnp.float32)
        acc2_scr[...] += r[:, :hh]
        out_scr[pl.ds(jr * cw, cw), :] += r[:, hh:hh + cc]

    @pl.when(ty == 1)
    def _b_diag():
        acc2_scr[...] += jnp.dot(q, u2_ref[pl.ds(b * cw, cw), :],
                                 preferred_element_type=jnp.float32)
        h2 = _layernorm(acc2_scr[...] + bh_ref[...], gh_ref[...],
                        beh_ref[...])
        h = jnp.maximum(h2, 0.0) + h1_ref[...]
        u3j = jnp.dot(h, Wout_ref[...],
                      preferred_element_type=jnp.float32).astype(jnp.bfloat16)
        u3_scr[pl.ds(jr * cw, cw), :] = u3j
        out_scr[pl.ds(jr * cw, cw), :] += jnp.dot(
            q, u3j, preferred_element_type=jnp.float32)

    @pl.when((ty == 2) | (ty == 3))
    def _c_acc():
        out_scr[pl.ds(jr * cw, cw), :] += jnp.dot(
            q, u3_scr[pl.ds(b * cw, cw), :],
            preferred_element_type=jnp.float32)

    @pl.when((ty == 3) | (ty == 4))
    def _finalize():
        o = _layernorm(out_scr[pl.ds(orow * cw, cw), :] + bout_ref[...],
                       gout_ref[...], beout_ref[...])
        out_ref[...] = o + skip_ref[...]


def kernel(x, adj, W_in, b_in, g_in, be_in, W_h, b_h, g_h, be_h,
           W_out, b_out, g_out, be_out, W_skip, b_skip):
    N, F = x.shape
    H = W_in.shape[1]
    C = W_out.shape[1]
    CW = N // NC
    BRA = CW // 10 if (CW % 10 == 0 and (CW // 10) % 8 == 0) else CW
    MB = NC
    _cp = pltpu.CompilerParams(vmem_limit_bytes=128 * 1024 * 1024)

    b_in2 = b_in.reshape(1, H)
    g_in2 = g_in.reshape(1, H)
    be_in2 = be_in.reshape(1, H)
    b_h2 = b_h.reshape(1, H)
    g_h2 = g_h.reshape(1, H)
    be_h2 = be_h.reshape(1, H)
    b_out2 = b_out.reshape(1, C)
    g_out2 = g_out.reshape(1, C)
    be_out2 = be_out.reshape(1, C)
    b_skip2 = b_skip.reshape(1, C)

    full = lambda shape: pl.BlockSpec(shape, lambda *a: (0,) * len(shape))

    u1, skip = pl.pallas_call(
        _prologue_kernel,
        grid=(1,),
        in_specs=[full((N, F)), full((F, H)), full((F, C)), full((1, C))],
        out_specs=[full((N, H)), full((N, C))],
        out_shape=[
            jax.ShapeDtypeStruct((N, H), jnp.bfloat16),
            jax.ShapeDtypeStruct((N, C), jnp.float32),
        ],
    )(x, W_in, W_skip, b_skip2)

    rowa = lambda w: pl.BlockSpec((BRA, w), lambda i: (i, 0))
    adjq, h1, u2, acc2p = pl.pallas_call(
        functools.partial(_phase_a_kernel, br=BRA, cw=CW),
        grid=(N // BRA,),
        in_specs=[rowa(N), full((N, H)), full((1, H)), full((1, H)),
                  full((1, H)), full((H, H))],
        out_specs=[pl.BlockSpec((NC, BRA, CW), lambda i: (0, i, 0)),
                   rowa(H), rowa(H), rowa(H)],
        out_shape=[
            jax.ShapeDtypeStruct((NC, N, CW), jnp.bfloat16),
            jax.ShapeDtypeStruct((N, H), jnp.float32),
            jax.ShapeDtypeStruct((N, H), jnp.bfloat16),
            jax.ShapeDtypeStruct((N, H), jnp.float32),
        ],
        scratch_shapes=[pltpu.VMEM((N, H), jnp.bfloat16),
                        pltpu.VMEM((CW, H), jnp.bfloat16)],
        compiler_params=_cp,
    )(adj, u1, b_in2, g_in2, be_in2, W_h)

    seq = []
    pending = []
    for j in range(MB - 1, -1, -1):
        bs = list(range(j + 1, MB)) + [j]
        for k, b in enumerate(bs):
            seq.append([j, b, 1 if b == j else 0, 1 if k == 0 else 0, j, -1])
            if k > 0 and pending:
                jc, bc = pending.pop(0)
                seq.append([jc, bc, 3 if bc == 0 else 2, 0, j,
                            jc if bc == 0 else -1])
        pending.extend((jc, j) for jc in range(j + 1, MB))
    for jc, bc in pending:
        seq.append([jc, bc, 3 if bc == 0 else 2, 0, 0, jc if bc == 0 else -1])
    seq.append([seq[-1][0], seq[-1][1], 4, 0, 0, 0])
    first_fin = next(r[5] for r in seq if r[5] >= 0)
    cur = first_fin
    for r in seq:
        if r[5] >= 0:
            cur = r[5]
        else:
            r[5] = cur
    sched = jnp.asarray(np.array(seq, dtype=np.int32).T)
    TT = len(seq)

    tile = pl.BlockSpec((1, CW, CW), lambda t, s: (s[1, t], s[0, t], 0))
    rowb = lambda w: pl.BlockSpec((CW, w), lambda t, s: (s[4, t], 0))
    rowo = lambda w: pl.BlockSpec((CW, w), lambda t, s: (s[5, t], 0))
    fullp = lambda shape: pl.BlockSpec(shape, lambda t, s: (0,) * len(shape))

    out = pl.pallas_call(
        functools.partial(_phase_bc_kernel, cw=CW, hh=H, cc=C),
        grid_spec=pltpu.PrefetchScalarGridSpec(
            num_scalar_prefetch=1,
            grid=(TT,),
            in_specs=[tile, fullp((N, H)), fullp((1, H)), fullp((1, H)),
                      fullp((1, H)), rowb(H), rowb(H), fullp((H, C)),
                      fullp((1, C)), fullp((1, C)), fullp((1, C)), rowo(C)],
            out_specs=rowo(C),
            scratch_shapes=[pltpu.VMEM((CW, H), jnp.float32),
                            pltpu.VMEM((N, C), jnp.bfloat16),
                            pltpu.VMEM((N, C), jnp.float32)],
        ),
        out_shape=jax.ShapeDtypeStruct((N, C), jnp.float32),
        compiler_params=_cp,
    )(sched, adjq, u2, b_h2, g_h2, be_h2, h1, acc2p, W_out,
      b_out2, g_out2, be_out2, skip)

    return out
